# baseline (speedup 1.0000x reference)
import functools

import jax
import jax.numpy as jnp
from jax import lax
from jax.experimental import pallas as pl
from jax.experimental.pallas import tpu as pltpu
from jax.experimental.pallas import tpu_sc as plsc

L = 16
NC = 2
NS = 16
NW = NC * NS


def _impl(x_flat, tt_flat, word_W, pos_W, tok_W, T):
    BT = x_flat.shape[0]
    H = word_W.shape[1]
    PER_W = BT // NW
    W = T
    NCH = PER_W // W
    WPAD = W + (-W) % L
    G0 = 128
    G1 = W - G0

    mesh = plsc.VectorSubcoreMesh(core_axis_name="c", subcore_axis_name="s")

    @functools.partial(
        pl.kernel,
        out_type=jax.ShapeDtypeStruct((BT, H), jnp.float32),
        mesh=mesh,
        scratch_types=[
            pltpu.VMEM((2, T, H), jnp.float32),
            pltpu.VMEM((2, H), jnp.float32),
            pltpu.VMEM((WPAD,), jnp.int32),
            pltpu.VMEM((WPAD,), jnp.int32),
            pltpu.VMEM((WPAD,), jnp.int32),
            pltpu.VMEM((WPAD,), jnp.int32),
            pltpu.VMEM((2, W, H), jnp.float32),
            pltpu.SemaphoreType.DMA,
            pltpu.SemaphoreType.DMA,
            pltpu.SemaphoreType.DMA,
            pltpu.SemaphoreType.DMA,
            pltpu.SemaphoreType.DMA,
            pltpu.SemaphoreType.DMA,
        ],
    )
    def k(x_hbm, tt_hbm, w_hbm, pos_hbm, tok_hbm, out_hbm,
          fused_v, tok_v, xidx_v0, xidx_v1, ttv_v0, ttv_v1, rows_v,
          sem_in0, sem_in1, sem_g0, sem_g1, sem_o0, sem_o1):
        wid = lax.axis_index("c") * NS + lax.axis_index("s")
        base = wid * PER_W
        xidx_v = [xidx_v0, xidx_v1]
        ttv_v = [ttv_v0, ttv_v1]
        sem_in = [sem_in0, sem_in1]
        sem_g = [sem_g0, sem_g1]
        sem_o = [sem_o0, sem_o1]

        pltpu.sync_copy(pos_hbm.at[pl.ds(0, T)], fused_v.at[0])
        pltpu.sync_copy(pos_hbm.at[pl.ds(0, T)], fused_v.at[1])
        pltpu.sync_copy(tok_hbm, tok_v)

        @pl.loop(0, T)
        def _(t):
            for j in range(H // L):
                sl = pl.ds(j * L, L)
                fused_v[0, t, sl] = fused_v[0, t, sl] + tok_v[0, sl]
                fused_v[1, t, sl] = fused_v[1, t, sl] + tok_v[1, sl]

        def start_in(g, buf):
            off = base + g * W
            pltpu.make_async_copy(
                x_hbm.at[pl.ds(off, W)],
                xidx_v[buf].at[pl.ds(0, W)], sem_in[buf]).start()
            pltpu.make_async_copy(
                tt_hbm.at[pl.ds(off, W)],
                ttv_v[buf].at[pl.ds(0, W)], sem_in[buf]).start()

        def wait_in(g, buf):
            off = base + g * W
            pltpu.make_async_copy(
                x_hbm.at[pl.ds(off, W)],
                xidx_v[buf].at[pl.ds(0, W)], sem_in[buf]).wait()
            pltpu.make_async_copy(
                tt_hbm.at[pl.ds(off, W)],
                ttv_v[buf].at[pl.ds(0, W)], sem_in[buf]).wait()

        def gather_copies(buf):
            return (
                pltpu.make_async_copy(
                    w_hbm.at[xidx_v[buf].at[pl.ds(0, G0)]],
                    rows_v.at[buf, pl.ds(0, G0)], sem_g[buf]),
                pltpu.make_async_copy(
                    w_hbm.at[xidx_v[buf].at[pl.ds(G0, G1)]],
                    rows_v.at[buf, pl.ds(G0, G1)], sem_g[buf]),
            )

        def start_gather(buf):
            for c in gather_copies(buf):
                c.start()

        def wait_gather(buf):
            for c in gather_copies(buf):
                c.wait()

        def start_out(g, buf):
            off = base + g * W
            pltpu.make_async_copy(
                rows_v.at[buf], out_hbm.at[pl.ds(off, W)], sem_o[buf]).start()

        def wait_out(g, buf):
            off = base + g * W
            pltpu.make_async_copy(
                rows_v.at[buf], out_hbm.at[pl.ds(off, W)], sem_o[buf]).wait()

        def row_add(buf, r, ttvec, i):
            tt = ttvec[i]
            for j in range(H // L):
                sl = pl.ds(j * L, L)
                plsc.addupdate(rows_v.at[buf, r, sl], fused_v[tt, r, sl])

        def compute(buf):
            @plsc.parallel_loop(0, W // L, unroll=2)
            def _(rg):
                r0 = rg * L
                ttvec = ttv_v[buf][pl.ds(r0, L)]
                for i in range(L):
                    row_add(buf, r0 + i, ttvec, i)

            rem = W % L
            if rem:
                r0 = W - rem
                ttvec = ttv_v[buf][pl.ds(r0, L)]
                for i in range(rem):
                    row_add(buf, r0 + i, ttvec, i)

        def step(g, buf):
            nbuf = 1 - buf
            wait_gather(buf)

            @pl.when(g + 1 < NCH)
            def _():
                wait_in(g + 1, nbuf)

                @pl.when(g >= 1)
                def _():
                    wait_out(g - 1, nbuf)

                start_gather(nbuf)

            compute(buf)
            start_out(g, buf)

            @pl.when(g + 2 < NCH)
            def _():
                start_in(g + 2, buf)

        start_in(0, 0)
        wait_in(0, 0)
        start_gather(0)
        start_in(1, 1)

        @pl.loop(0, NCH, step=2)
        def _(c0):
            step(c0, 0)
            step(c0 + 1, 1)

        wait_out(NCH - 1, 1)

    return k(x_flat, tt_flat, word_W, pos_W, tok_W)


def kernel(x, token_type, word_W, pos_W, tok_W):
    B, T = x.shape
    H = word_W.shape[1]
    out = _impl(x.reshape(-1), token_type.reshape(-1), word_W, pos_W, tok_W, T)
    return out.reshape(B, T, H)


# speedup vs baseline: 1.8917x; 1.8917x over previous
import functools

import jax
import jax.numpy as jnp
from jax import lax
from jax.experimental import pallas as pl
from jax.experimental.pallas import tpu as pltpu
from jax.experimental.pallas import tpu_sc as plsc

L = 16
NC = 2
NS = 16
NW = NC * NS


def _impl(x_flat, tt_flat, word_W, pos_W, tok_W, T):
    BT = x_flat.shape[0]
    H = word_W.shape[1]
    PER_W = BT // NW
    W = T
    NCH = PER_W // W
    WPAD = W + L
    G0 = 128
    G1 = W - G0

    mesh = plsc.VectorSubcoreMesh(core_axis_name="c", subcore_axis_name="s")

    @functools.partial(
        pl.kernel,
        out_type=jax.ShapeDtypeStruct((BT, H), jnp.float32),
        mesh=mesh,
        scratch_types=[
            pltpu.VMEM((2, T, H), jnp.float32),
            pltpu.VMEM((2, H), jnp.float32),
            pltpu.VMEM((WPAD,), jnp.int32),
            pltpu.VMEM((WPAD,), jnp.int32),
            pltpu.VMEM((WPAD,), jnp.int32),
            pltpu.VMEM((WPAD,), jnp.int32),
            pltpu.VMEM((2, W, H), jnp.float32),
            pltpu.SemaphoreType.DMA,
            pltpu.SemaphoreType.DMA,
            pltpu.SemaphoreType.DMA,
            pltpu.SemaphoreType.DMA,
            pltpu.SemaphoreType.DMA,
            pltpu.SemaphoreType.DMA,
        ],
    )
    def k(x_hbm, tt_hbm, w_hbm, pos_hbm, tok_hbm, out_hbm,
          fused_v, tok_v, xidx_v0, xidx_v1, ttv_v0, ttv_v1, rows_v,
          sem_in0, sem_in1, sem_g0, sem_g1, sem_o0, sem_o1):
        wid = lax.axis_index("c") * NS + lax.axis_index("s")
        base = wid * PER_W
        xidx_v = [xidx_v0, xidx_v1]
        ttv_v = [ttv_v0, ttv_v1]
        sem_in = [sem_in0, sem_in1]
        sem_g = [sem_g0, sem_g1]
        sem_o = [sem_o0, sem_o1]

        pltpu.sync_copy(pos_hbm.at[pl.ds(0, T)], fused_v.at[0])
        pltpu.sync_copy(pos_hbm.at[pl.ds(0, T)], fused_v.at[1])
        pltpu.sync_copy(tok_hbm, tok_v)

        @pl.loop(0, T)
        def _(t):
            for j in range(H // L):
                sl = pl.ds(j * L, L)
                fused_v[0, t, sl] = fused_v[0, t, sl] + tok_v[0, sl]
                fused_v[1, t, sl] = fused_v[1, t, sl] + tok_v[1, sl]

        def start_in(g, buf):
            off = base + g * W
            pltpu.make_async_copy(
                x_hbm.at[pl.ds(off, W)],
                xidx_v[buf].at[pl.ds(0, W)], sem_in[buf]).start()
            pltpu.make_async_copy(
                tt_hbm.at[pl.ds(off, W)],
                ttv_v[buf].at[pl.ds(0, W)], sem_in[buf]).start()

        def wait_in(g, buf):
            off = base + g * W
            pltpu.make_async_copy(
                x_hbm.at[pl.ds(off, W)],
                xidx_v[buf].at[pl.ds(0, W)], sem_in[buf]).wait()
            pltpu.make_async_copy(
                tt_hbm.at[pl.ds(off, W)],
                ttv_v[buf].at[pl.ds(0, W)], sem_in[buf]).wait()

        def gather_copies(buf):
            return (
                pltpu.make_async_copy(
                    w_hbm.at[xidx_v[buf].at[pl.ds(0, G0)]],
                    rows_v.at[buf, pl.ds(0, G0)], sem_g[buf]),
                pltpu.make_async_copy(
                    w_hbm.at[xidx_v[buf].at[pl.ds(G0, G1)]],
                    rows_v.at[buf, pl.ds(G0, G1)], sem_g[buf]),
            )

        def start_gather(buf):
            for c in gather_copies(buf):
                c.start()

        def wait_gather(buf):
            for c in gather_copies(buf):
                c.wait()

        def start_out(g, buf):
            off = base + g * W
            pltpu.make_async_copy(
                rows_v.at[buf], out_hbm.at[pl.ds(off, W)], sem_o[buf]).start()

        def wait_out(g, buf):
            off = base + g * W
            pltpu.make_async_copy(
                rows_v.at[buf], out_hbm.at[pl.ds(off, W)], sem_o[buf]).wait()

        def compute(buf):
            @plsc.parallel_loop(0, W, unroll=4)
            def _(r):
                ttvec = ttv_v[buf][pl.ds(r, L)]
                tt = ttvec[0]
                for j in range(H // L):
                    sl = pl.ds(j * L, L)
                    plsc.addupdate(rows_v.at[buf, r, sl], fused_v[tt, r, sl])

        def step(g, buf):
            nbuf = 1 - buf
            wait_gather(buf)

            @pl.when(g + 1 < NCH)
            def _():
                wait_in(g + 1, nbuf)

                @pl.when(g >= 1)
                def _():
                    wait_out(g - 1, nbuf)

                start_gather(nbuf)

            compute(buf)
            start_out(g, buf)

            @pl.when(g + 2 < NCH)
            def _():
                start_in(g + 2, buf)

        start_in(0, 0)
        wait_in(0, 0)
        start_gather(0)
        start_in(1, 1)

        @pl.loop(0, NCH, step=2)
        def _(c0):
            step(c0, 0)
            step(c0 + 1, 1)

        wait_out(NCH - 1, 1)

    return k(x_flat, tt_flat, word_W, pos_W, tok_W)


def kernel(x, token_type, word_W, pos_W, tok_W):
    B, T = x.shape
    H = word_W.shape[1]
    out = _impl(x.reshape(-1), token_type.reshape(-1), word_W, pos_W, tok_W, T)
    return out.reshape(B, T, H)


# speedup vs baseline: 1.8945x; 1.0014x over previous
import functools

import jax
import jax.numpy as jnp
from jax import lax
from jax.experimental import pallas as pl
from jax.experimental.pallas import tpu as pltpu
from jax.experimental.pallas import tpu_sc as plsc

L = 16
NC = 2
NS = 16
NW = NC * NS


def _impl(x_flat, tt_flat, word_W, pos_W, tok_W, T):
    BT = x_flat.shape[0]
    H = word_W.shape[1]
    PER_W = BT // NW
    W = T
    NCH = PER_W // W
    WPAD = W + L
    G0 = 128
    G1 = W - G0

    mesh = plsc.VectorSubcoreMesh(core_axis_name="c", subcore_axis_name="s")

    @functools.partial(
        pl.kernel,
        out_type=jax.ShapeDtypeStruct((BT, H), jnp.float32),
        mesh=mesh,
        scratch_types=[
            pltpu.VMEM((2, T, H), jnp.float32),
            pltpu.VMEM((2, H), jnp.float32),
            pltpu.VMEM((WPAD,), jnp.int32),
            pltpu.VMEM((WPAD,), jnp.int32),
            pltpu.VMEM((WPAD,), jnp.int32),
            pltpu.VMEM((WPAD,), jnp.int32),
            pltpu.VMEM((2, W, H), jnp.float32),
            pltpu.SemaphoreType.DMA,
            pltpu.SemaphoreType.DMA,
            pltpu.SemaphoreType.DMA,
            pltpu.SemaphoreType.DMA,
            pltpu.SemaphoreType.DMA,
            pltpu.SemaphoreType.DMA,
        ],
    )
    def k(x_hbm, tt_hbm, w_hbm, pos_hbm, tok_hbm, out_hbm,
          fused_v, tok_v, xidx_v0, xidx_v1, ttv_v0, ttv_v1, rows_v,
          sem_in0, sem_in1, sem_g0, sem_g1, sem_o0, sem_o1):
        wid = lax.axis_index("c") * NS + lax.axis_index("s")
        base = wid * PER_W
        xidx_v = [xidx_v0, xidx_v1]
        ttv_v = [ttv_v0, ttv_v1]
        sem_in = [sem_in0, sem_in1]
        sem_g = [sem_g0, sem_g1]
        sem_o = [sem_o0, sem_o1]

        def start_in(g, buf):
            off = base + g * W
            pltpu.make_async_copy(
                x_hbm.at[pl.ds(off, W)],
                xidx_v[buf].at[pl.ds(0, W)], sem_in[buf]).start()
            pltpu.make_async_copy(
                tt_hbm.at[pl.ds(off, W)],
                ttv_v[buf].at[pl.ds(0, W)], sem_in[buf]).start()

        def wait_in(g, buf):
            off = base + g * W
            pltpu.make_async_copy(
                x_hbm.at[pl.ds(off, W)],
                xidx_v[buf].at[pl.ds(0, W)], sem_in[buf]).wait()
            pltpu.make_async_copy(
                tt_hbm.at[pl.ds(off, W)],
                ttv_v[buf].at[pl.ds(0, W)], sem_in[buf]).wait()

        def gather_copies(buf):
            return (
                pltpu.make_async_copy(
                    w_hbm.at[xidx_v[buf].at[pl.ds(0, G0)]],
                    rows_v.at[buf, pl.ds(0, G0)], sem_g[buf]),
                pltpu.make_async_copy(
                    w_hbm.at[xidx_v[buf].at[pl.ds(G0, G1)]],
                    rows_v.at[buf, pl.ds(G0, G1)], sem_g[buf]),
            )

        def start_gather(buf):
            for c in gather_copies(buf):
                c.start()

        def wait_gather(buf):
            for c in gather_copies(buf):
                c.wait()

        def start_out(g, buf):
            off = base + g * W
            pltpu.make_async_copy(
                rows_v.at[buf], out_hbm.at[pl.ds(off, W)], sem_o[buf]).start()

        def wait_out(g, buf):
            off = base + g * W
            pltpu.make_async_copy(
                rows_v.at[buf], out_hbm.at[pl.ds(off, W)], sem_o[buf]).wait()

        def compute(buf):
            @plsc.parallel_loop(0, W, unroll=4)
            def _(r):
                ttvec = ttv_v[buf][pl.ds(r, L)]
                tt = ttvec[0]
                for j in range(H // L):
                    sl = pl.ds(j * L, L)
                    plsc.addupdate(rows_v.at[buf, r, sl], fused_v[tt, r, sl])

        def step(g, buf):
            nbuf = 1 - buf
            wait_gather(buf)

            @pl.when(g + 1 < NCH)
            def _():
                wait_in(g + 1, nbuf)

                @pl.when(g >= 1)
                def _():
                    wait_out(g - 1, nbuf)

                start_gather(nbuf)

            compute(buf)
            start_out(g, buf)

            @pl.when(g + 2 < NCH)
            def _():
                start_in(g + 2, buf)

        start_in(0, 0)
        wait_in(0, 0)
        start_gather(0)
        start_in(1, 1)

        pltpu.sync_copy(pos_hbm.at[pl.ds(0, T)], fused_v.at[0])
        pltpu.sync_copy(pos_hbm.at[pl.ds(0, T)], fused_v.at[1])
        pltpu.sync_copy(tok_hbm, tok_v)

        @pl.loop(0, T)
        def _(t):
            for j in range(H // L):
                sl = pl.ds(j * L, L)
                fused_v[0, t, sl] = fused_v[0, t, sl] + tok_v[0, sl]
                fused_v[1, t, sl] = fused_v[1, t, sl] + tok_v[1, sl]


        @pl.loop(0, NCH, step=2)
        def _(c0):
            step(c0, 0)
            step(c0 + 1, 1)

        wait_out(NCH - 1, 1)

    return k(x_flat, tt_flat, word_W, pos_W, tok_W)


def kernel(x, token_type, word_W, pos_W, tok_W):
    B, T = x.shape
    H = word_W.shape[1]
    out = _impl(x.reshape(-1), token_type.reshape(-1), word_W, pos_W, tok_W, T)
    return out.reshape(B, T, H)
